# baseline (speedup 1.0000x reference)
import functools

import jax
import jax.numpy as jnp
import numpy as np
from jax import lax
from jax.experimental import pallas as pl
from jax.experimental.pallas import tpu as pltpu
from jax.experimental.pallas import tpu_sc as plsc

_DEPTH = 64
_SEQ = 200
_NC, _NS, _L = 2, 16, 16
_NW = _NC * _NS
_CHUNK = _SEQ


def _pos_encoding(length, depth, n=10000):
    positions = np.arange(length)[:, np.newaxis]
    depths = np.arange(depth)[np.newaxis, :] / depth
    angle_rates = 1 / n**depths
    angle_rads = positions * angle_rates
    angle_rads[:, 0::2] = np.sin(angle_rads[:, 0::2])
    angle_rads[:, 1::2] = np.cos(angle_rads[:, 1::2])
    return angle_rads.astype(np.float32)


_POS = _pos_encoding(_SEQ, _DEPTH)


def _make_sc_kernel(n_rows):
    rows_per_w = n_rows // _NW
    n_chunks = rows_per_w // _CHUNK
    mesh = plsc.VectorSubcoreMesh(
        core_axis_name="c", subcore_axis_name="s", num_cores=_NC,
        num_subcores=_NS)

    @functools.partial(
        pl.kernel,
        out_type=jax.ShapeDtypeStruct((n_rows, _DEPTH), jnp.float32),
        mesh=mesh,
        scratch_types=[
            pltpu.VMEM((rows_per_w,), jnp.int32),
            pltpu.VMEM((_SEQ, _DEPTH), jnp.float32),
            pltpu.VMEM((_CHUNK, _DEPTH), jnp.float32),
            pltpu.SemaphoreType.DMA,
        ],
        compiler_params=pltpu.CompilerParams(use_tc_tiling_on_sc=False),
    )
    def k(x_hbm, table_hbm, pos_hbm, out_hbm, idxs, posb, rows, gsem):
        wid = lax.axis_index("s") * _NC + lax.axis_index("c")
        base = wid * rows_per_w
        pltpu.sync_copy(x_hbm.at[pl.ds(base, rows_per_w)], idxs)
        pltpu.sync_copy(pos_hbm, posb)

        @pl.loop(0, n_chunks)
        def _chunk(c):
            off = c * _CHUNK
            pltpu.async_copy(
                table_hbm.at[idxs.at[pl.ds(off, _CHUNK)]], rows, gsem
            ).wait()

            @plsc.parallel_loop(0, _CHUNK, 1, unroll=2)
            def _row(i):
                for d in range(_DEPTH // _L):
                    sl = pl.ds(d * _L, _L)
                    rows[i, sl] = rows[i, sl] * 8.0 + posb[i, sl]

            pltpu.sync_copy(rows, out_hbm.at[pl.ds(base + off, _CHUNK)])

    return k


@jax.jit
def kernel(x, table):
    b, s = x.shape
    x_flat = x.reshape(b * s).astype(jnp.int32)
    pos = jnp.asarray(_POS)
    out = _make_sc_kernel(b * s)(x_flat, table, pos)
    return out.reshape(b, s, _DEPTH)


# speedup vs baseline: 1.0638x; 1.0638x over previous
import functools

import jax
import jax.numpy as jnp
import numpy as np
from jax import lax
from jax.experimental import pallas as pl
from jax.experimental.pallas import tpu as pltpu
from jax.experimental.pallas import tpu_sc as plsc

_DEPTH = 64
_SEQ = 200
_NC, _NS, _L = 2, 16, 16
_NW = _NC * _NS
_CHUNK = _SEQ
_NBUF = 4


def _pos_encoding(length, depth, n=10000):
    positions = np.arange(length)[:, np.newaxis]
    depths = np.arange(depth)[np.newaxis, :] / depth
    angle_rates = 1 / n**depths
    angle_rads = positions * angle_rates
    angle_rads[:, 0::2] = np.sin(angle_rads[:, 0::2])
    angle_rads[:, 1::2] = np.cos(angle_rads[:, 1::2])
    return angle_rads.astype(np.float32)


_POS = _pos_encoding(_SEQ, _DEPTH)


def _make_sc_kernel(n_rows):
    rows_per_w = n_rows // _NW
    n_chunks = rows_per_w // _CHUNK
    mesh = plsc.VectorSubcoreMesh(
        core_axis_name="c", subcore_axis_name="s", num_cores=_NC,
        num_subcores=_NS)

    @functools.partial(
        pl.kernel,
        out_type=jax.ShapeDtypeStruct((n_rows, _DEPTH), jnp.float32),
        mesh=mesh,
        scratch_types=[
            pltpu.VMEM((rows_per_w,), jnp.int32),
            pltpu.VMEM((_SEQ, _DEPTH), jnp.float32),
            pltpu.VMEM((_NBUF, _CHUNK, _DEPTH), jnp.float32),
        ] + [pltpu.SemaphoreType.DMA] * (2 * _NBUF),
        compiler_params=pltpu.CompilerParams(use_tc_tiling_on_sc=False),
    )
    def k(x_hbm, table_hbm, pos_hbm, out_hbm, idxs, posb, rows, *sems):
        gsems, osems = sems[:_NBUF], sems[_NBUF:]
        wid = lax.axis_index("s") * _NC + lax.axis_index("c")
        base = wid * rows_per_w
        pltpu.sync_copy(x_hbm.at[pl.ds(base, rows_per_w)], idxs)
        pltpu.sync_copy(pos_hbm, posb)

        def gather_start(c, b):
            pltpu.async_copy(
                table_hbm.at[idxs.at[pl.ds(c * _CHUNK, _CHUNK)]],
                rows.at[b], gsems[b])

        def gather_wait(b):
            pltpu.make_async_copy(
                out_hbm.at[pl.ds(0, _CHUNK)], rows.at[b], gsems[b]).wait()

        def scatter_wait(b):
            pltpu.make_async_copy(
                rows.at[b], out_hbm.at[pl.ds(0, _CHUNK)], osems[b]).wait()

        for b in range(_NBUF - 1):
            gather_start(b, b)

        @pl.loop(0, n_chunks, step=_NBUF)
        def _chunks(c0):
            for b in range(_NBUF):
                c = c0 + b
                gather_wait(b)

                @plsc.parallel_loop(0, _CHUNK, 1, unroll=2)
                def _row(i):
                    for d in range(_DEPTH // _L):
                        sl = pl.ds(d * _L, _L)
                        rows[b, i, sl] = rows[b, i, sl] * 8.0 + posb[i, sl]

                pltpu.async_copy(
                    rows.at[b],
                    out_hbm.at[pl.ds(base + c * _CHUNK, _CHUNK)], osems[b])

                nc = c + _NBUF - 1
                bb = (b + _NBUF - 1) % _NBUF
                @pl.when(nc < n_chunks)
                def _():
                    @pl.when(nc >= _NBUF)
                    def _():
                        scatter_wait(bb)
                    gather_start(nc, bb)

        for b in range(_NBUF):
            scatter_wait(b)

    return k


@jax.jit
def kernel(x, table):
    b, s = x.shape
    x_flat = x.reshape(b * s).astype(jnp.int32)
    pos = jnp.asarray(_POS)
    out = _make_sc_kernel(b * s)(x_flat, table, pos)
    return out.reshape(b, s, _DEPTH)
